# baseline (speedup 1.0000x reference)
import functools

import jax
import jax.numpy as jnp
from jax.experimental import pallas as pl
from jax.experimental.pallas import tpu as pltpu

B = 32768
DS = 658
DB = 250
H = 256
A = 20
BM = 4096
OUT_W = 128
_MM_DTYPE = jnp.bfloat16


def _body(xst_ref, xb_ref, act_ref, wst_ref, wbt_ref, b1_ref, w2t_ref,
          b2_ref, lp_ref, val_ref, ent_ref):
    bf = _MM_DTYPE
    acc = jax.lax.dot_general(
        wst_ref[...].astype(bf), xst_ref[...].astype(bf),
        dimension_numbers=(((0,), (0,)), ((), ())),
        preferred_element_type=jnp.float32)
    acc = acc + jax.lax.dot_general(
        wbt_ref[...].astype(bf), xb_ref[...].astype(bf),
        dimension_numbers=(((0,), (1,)), ((), ())),
        preferred_element_type=jnp.float32)
    ht = jnp.tanh(acc + b1_ref[...])
    o2t = jax.lax.dot_general(
        w2t_ref[...].astype(bf), ht.astype(bf),
        dimension_numbers=(((0,), (0,)), ((), ())),
        preferred_element_type=jnp.float32) + b2_ref[...]
    logits = o2t[:A]
    value = o2t[A:A + 1]
    m = jnp.max(logits, axis=0, keepdims=True)
    e = jnp.exp(logits - m)
    z = jnp.sum(e, axis=0, keepdims=True)
    logz = jnp.log(z)
    s = jnp.sum(e * (logits - m), axis=0, keepdims=True)
    ent = logz - s / z
    onehot = jax.lax.broadcasted_iota(jnp.int32, logits.shape, 0) == act_ref[...]
    g = jnp.sum(jnp.where(onehot, logits, 0.0), axis=0, keepdims=True)
    alp = g - m - logz
    lp_ref[...] = alp
    val_ref[...] = value
    ent_ref[...] = ent


@functools.partial(jax.jit, static_argnames=("interpret",))
def _run(states, believes, actions, Wa1, ba1, Wa2, ba2, Wc1, bc1, Wc2, bc2,
         interpret=False):
    wst = jnp.concatenate([Wa1[:DS], Wc1], axis=1)
    wbt = jnp.concatenate([Wa1[DS:], jnp.zeros((DB, H), jnp.float32)], axis=1)
    b1 = jnp.concatenate([ba1, bc1])[:, None]
    w2t = jnp.zeros((2 * H, OUT_W), jnp.float32)
    w2t = w2t.at[:H, :A].set(Wa2).at[H:, A].set(Wc2[:, 0])
    b2 = jnp.zeros((OUT_W, 1), jnp.float32).at[:A, 0].set(ba2).at[A, 0].set(bc2[0])
    statest = states.T
    act2d = actions.astype(jnp.int32)[None, :]

    grid = (B // BM,)
    out = pl.pallas_call(
        _body,
        grid=grid,
        in_specs=[
            pl.BlockSpec((DS, BM), lambda i: (0, i)),
            pl.BlockSpec((BM, DB), lambda i: (i, 0)),
            pl.BlockSpec((1, BM), lambda i: (0, i)),
            pl.BlockSpec((DS, 2 * H), lambda i: (0, 0)),
            pl.BlockSpec((DB, 2 * H), lambda i: (0, 0)),
            pl.BlockSpec((2 * H, 1), lambda i: (0, 0)),
            pl.BlockSpec((2 * H, OUT_W), lambda i: (0, 0)),
            pl.BlockSpec((OUT_W, 1), lambda i: (0, 0)),
        ],
        out_specs=[
            pl.BlockSpec((1, BM), lambda i: (0, i)),
            pl.BlockSpec((1, BM), lambda i: (0, i)),
            pl.BlockSpec((1, BM), lambda i: (0, i)),
        ],
        out_shape=[jax.ShapeDtypeStruct((1, B), jnp.float32)] * 3,
        compiler_params=pltpu.CompilerParams(
            dimension_semantics=("parallel",)),
        interpret=interpret,
    )(statest, believes, act2d, wst, wbt, b1, w2t, b2)
    return out[0][0], out[1][0], out[2][0]


def kernel(states, believes, actions, Wa1, ba1, Wa2, ba2, Wc1, bc1, Wc2, bc2):
    return _run(states, believes, actions, Wa1, ba1, Wa2, ba2,
                Wc1, bc1, Wc2, bc2)


# speedup vs baseline: 1.0346x; 1.0346x over previous
import functools

import jax
import jax.numpy as jnp
from jax.experimental import pallas as pl
from jax.experimental.pallas import tpu as pltpu

B = 32768
DS = 658
DB = 250
H = 256
A = 20
BM = 4096
_MM_DTYPE = jnp.bfloat16


def _body(xst_ref, xb_ref, act_ref, wa1_ref, ba1_ref, wa2_ref, ba2_ref,
          wc1_ref, bc1_ref, wc2_ref, bc2_ref, lp_ref, val_ref, ent_ref):
    bf = _MM_DTYPE
    xst = xst_ref[...].astype(bf)
    wa1 = wa1_ref[...].astype(bf)
    cdn_s = (((0,), (0,)), ((), ()))
    ha = jax.lax.dot_general(wa1[:DS], xst, cdn_s,
                             preferred_element_type=jnp.float32)
    ha = ha + jax.lax.dot_general(
        wa1[DS:], xb_ref[...].astype(bf),
        dimension_numbers=(((0,), (1,)), ((), ())),
        preferred_element_type=jnp.float32)
    ha = jnp.tanh(ha + ba1_ref[...])
    hc = jax.lax.dot_general(wc1_ref[...].astype(bf), xst, cdn_s,
                             preferred_element_type=jnp.float32)
    hc = jnp.tanh(hc + bc1_ref[...])
    logits = jax.lax.dot_general(
        wa2_ref[...].astype(bf), ha.astype(bf), cdn_s,
        preferred_element_type=jnp.float32) + ba2_ref[...]
    value = jax.lax.dot_general(
        wc2_ref[...].astype(bf), hc.astype(bf), cdn_s,
        preferred_element_type=jnp.float32) + bc2_ref[...]
    m = jnp.max(logits, axis=0, keepdims=True)
    e = jnp.exp(logits - m)
    z = jnp.sum(e, axis=0, keepdims=True)
    logz = jnp.log(z)
    s = jnp.sum(e * (logits - m), axis=0, keepdims=True)
    ent = logz - s / z
    onehot = jax.lax.broadcasted_iota(jnp.int32, logits.shape, 0) == act_ref[...]
    g = jnp.sum(jnp.where(onehot, logits, 0.0), axis=0, keepdims=True)
    alp = g - m - logz
    lp_ref[...] = alp
    val_ref[...] = value
    ent_ref[...] = ent


@functools.partial(jax.jit, static_argnames=("interpret",))
def _run(states, believes, actions, Wa1, ba1, Wa2, ba2, Wc1, bc1, Wc2, bc2,
         interpret=False):
    statest = states.T
    act2d = actions.astype(jnp.int32)[None, :]
    ba1c = ba1[:, None]
    ba2c = ba2[:, None]
    bc1c = bc1[:, None]
    bc2c = bc2[:, None]

    grid = (B // BM,)
    full = lambda i: (0, 0)
    out = pl.pallas_call(
        _body,
        grid=grid,
        in_specs=[
            pl.BlockSpec((DS, BM), lambda i: (0, i)),
            pl.BlockSpec((BM, DB), lambda i: (i, 0)),
            pl.BlockSpec((1, BM), lambda i: (0, i)),
            pl.BlockSpec((DS + DB, H), full),
            pl.BlockSpec((H, 1), full),
            pl.BlockSpec((H, A), full),
            pl.BlockSpec((A, 1), full),
            pl.BlockSpec((DS, H), full),
            pl.BlockSpec((H, 1), full),
            pl.BlockSpec((H, 1), full),
            pl.BlockSpec((1, 1), full),
        ],
        out_specs=[
            pl.BlockSpec((1, BM), lambda i: (0, i)),
            pl.BlockSpec((1, BM), lambda i: (0, i)),
            pl.BlockSpec((1, BM), lambda i: (0, i)),
        ],
        out_shape=[jax.ShapeDtypeStruct((1, B), jnp.float32)] * 3,
        compiler_params=pltpu.CompilerParams(
            dimension_semantics=("parallel",)),
        interpret=interpret,
    )(statest, believes, act2d, Wa1, ba1c, Wa2, ba2c, Wc1, bc1c, Wc2, bc2c)
    return out[0][0], out[1][0], out[2][0]


def kernel(states, believes, actions, Wa1, ba1, Wa2, ba2, Wc1, bc1, Wc2, bc2):
    return _run(states, believes, actions, Wa1, ba1, Wa2, ba2,
                Wc1, bc1, Wc2, bc2)
